# baseline (speedup 1.0000x reference)
import jax
import jax.numpy as jnp
from jax.experimental import pallas as pl
from jax.experimental.pallas import tpu as pltpu

_B, _SEG, _D, _H, _O = 16, 2048, 1024, 256, 32
_C = 2
_CH = _SEG // _C


def _readout_body(x_ref, w1_ref, w2_ref, o_ref, l_ref):
    c = pl.program_id(1)
    w2 = w2_ref[...]
    w1b = w1_ref[...].astype(jnp.bfloat16)
    w2b = w2.astype(jnp.bfloat16)
    k = jnp.sum(jnp.abs(w2), axis=1)
    x = x_ref[...]
    xb = x.astype(jnp.bfloat16)
    t = jnp.tanh(jnp.dot(xb, w1b.T, preferred_element_type=jnp.float32))
    s = jnp.dot(t.astype(jnp.bfloat16), w2b.T,
                preferred_element_type=jnp.float32)
    e = jnp.exp(s - k[None, :])
    part_l = jnp.sum(e, axis=0, keepdims=True)
    acc = jax.lax.dot_general(
        e.astype(jnp.bfloat16), xb, (((0,), (0,)), ((), ())),
        preferred_element_type=jnp.float32)

    @pl.when(c == 0)
    def _init():
        o_ref[...] = acc
        l_ref[...] = part_l

    @pl.when(c > 0)
    def _accum():
        o_ref[...] += acc
        l_ref[...] += part_l

    @pl.when(c == _C - 1)
    def _normalize():
        o_ref[...] = o_ref[...] / l_ref[0, :][:, None]


def kernel(embeddings, scope, w1, w2):
    del scope
    out = pl.pallas_call(
        _readout_body,
        grid=(_B, _C),
        in_specs=[
            pl.BlockSpec((_CH, _D), lambda b, c: (b * _C + c, 0)),
            pl.BlockSpec((_H, _D), lambda b, c: (0, 0)),
            pl.BlockSpec((_O, _H), lambda b, c: (0, 0)),
        ],
        out_specs=pl.BlockSpec((_O, _D), lambda b, c: (b, 0)),
        out_shape=jax.ShapeDtypeStruct((_B * _O, _D), jnp.float32),
        scratch_shapes=[pltpu.VMEM((1, _O), jnp.float32)],
    )(embeddings, w1, w2)
    return out.reshape(_B, _O * _D)


# speedup vs baseline: 1.1795x; 1.1795x over previous
import jax
import jax.numpy as jnp
from jax.experimental import pallas as pl
from jax.experimental.pallas import tpu as pltpu

_B, _SEG, _D, _H, _O = 16, 2048, 1024, 256, 32


def _readout_body(x_ref, w1_ref, w2_ref, o_ref):
    x = x_ref[...]
    xb = x.astype(jnp.bfloat16)
    w2 = w2_ref[...]
    t = jnp.tanh(jnp.dot(xb, w1_ref[...].astype(jnp.bfloat16).T,
                         preferred_element_type=jnp.float32))
    s = jnp.dot(t.astype(jnp.bfloat16), w2.astype(jnp.bfloat16).T,
                preferred_element_type=jnp.float32)
    k = jnp.sum(jnp.abs(w2), axis=1)
    e = jnp.exp(s - k[None, :])
    l = jnp.sum(e, axis=0)
    acc = jax.lax.dot_general(
        e.astype(jnp.bfloat16), xb, (((0,), (0,)), ((), ())),
        preferred_element_type=jnp.float32)
    o_ref[...] = acc / l[:, None]


def kernel(embeddings, scope, w1, w2):
    del scope
    out = pl.pallas_call(
        _readout_body,
        grid=(_B,),
        in_specs=[
            pl.BlockSpec((_SEG, _D), lambda b: (b, 0)),
            pl.BlockSpec((_H, _D), lambda b: (0, 0)),
            pl.BlockSpec((_O, _H), lambda b: (0, 0)),
        ],
        out_specs=pl.BlockSpec((_O, _D), lambda b: (b, 0)),
        out_shape=jax.ShapeDtypeStruct((_B * _O, _D), jnp.float32),
        compiler_params=pltpu.CompilerParams(
            dimension_semantics=("parallel",)),
    )(embeddings, w1, w2)
    return out.reshape(_B, _O * _D)


# speedup vs baseline: 1.1861x; 1.0055x over previous
import jax
import jax.numpy as jnp
from jax.experimental import pallas as pl
from jax.experimental.pallas import tpu as pltpu

_B, _SEG, _D, _H, _O = 16, 2048, 1024, 256, 32


_DH = _D // 2


def _readout_body(xl_ref, xr_ref, w1_ref, w2_ref, o_ref):
    xlb = xl_ref[...].astype(jnp.bfloat16)
    xrb = xr_ref[...].astype(jnp.bfloat16)
    w1b = w1_ref[...].astype(jnp.bfloat16)
    w2 = w2_ref[...]
    t = jnp.tanh(
        jnp.dot(xlb, w1b[:, :_DH].T, preferred_element_type=jnp.float32)
        + jnp.dot(xrb, w1b[:, _DH:].T, preferred_element_type=jnp.float32))
    s = jnp.dot(t.astype(jnp.bfloat16), w2.astype(jnp.bfloat16).T,
                preferred_element_type=jnp.float32)
    k = jnp.sum(jnp.abs(w2), axis=1)
    e = jnp.exp(s - k[None, :])
    l = jnp.sum(e, axis=0)
    eb = e.astype(jnp.bfloat16)
    accl = jax.lax.dot_general(
        eb, xlb, (((0,), (0,)), ((), ())), preferred_element_type=jnp.float32)
    accr = jax.lax.dot_general(
        eb, xrb, (((0,), (0,)), ((), ())), preferred_element_type=jnp.float32)
    rl = l[:, None]
    o_ref[:, :_DH] = accl / rl
    o_ref[:, _DH:] = accr / rl


def kernel(embeddings, scope, w1, w2):
    del scope
    out = pl.pallas_call(
        _readout_body,
        grid=(_B,),
        in_specs=[
            pl.BlockSpec((_SEG, _DH), lambda b: (b, 0)),
            pl.BlockSpec((_SEG, _DH), lambda b: (b, 1)),
            pl.BlockSpec((_H, _D), lambda b: (0, 0)),
            pl.BlockSpec((_O, _H), lambda b: (0, 0)),
        ],
        out_specs=pl.BlockSpec((_O, _D), lambda b: (b, 0)),
        out_shape=jax.ShapeDtypeStruct((_B * _O, _D), jnp.float32),
        compiler_params=pltpu.CompilerParams(
            dimension_semantics=("parallel",)),
    )(embeddings, embeddings, w1, w2)
    return out.reshape(_B, _O * _D)
